# baseline (speedup 1.0000x reference)
import functools

import jax
import jax.numpy as jnp
from jax import lax
from jax.experimental import pallas as pl
from jax.experimental.pallas import tpu as pltpu
from jax.experimental.pallas import tpu_sc as plsc

S = 2048
EM = 1024
NE = 8
HID = 4096
T = 256
_TSHIFT = 8
CH_T = 4
XSPAD = (CH_T - 1) * T
NTT = S * 2 + NE * T
NT = NTT // T
NT_PAD = 48
NW = 32
TPW = S // NW
CHT = 16


_TOK_BLK = 256


_GPB = _TOK_BLK // TPW


def _router_body(x_ref, wr_ref, br_ref, idx_ref, wt_ref, cnt_ref):
    x = x_ref[...]
    wr = wr_ref[...]
    scores = lax.dot_general(x, wr, (((1,), (1,)), ((), ())),
                             preferred_element_type=jnp.float32)
    scores = scores + br_ref[...]
    ii = lax.broadcasted_iota(jnp.int32, (_TOK_BLK, NE), 1)
    m1 = jnp.max(scores, axis=1, keepdims=True)
    i1 = jnp.min(jnp.where(scores == m1, ii, NE), axis=1, keepdims=True)
    masked = jnp.where(ii == i1, -jnp.inf, scores)
    m2 = jnp.max(masked, axis=1, keepdims=True)
    i2 = jnp.min(jnp.where(masked == m2, ii, NE), axis=1, keepdims=True)
    e2 = jnp.exp(m2 - m1)
    w1 = 1.0 / (1.0 + e2)
    w2 = 1.0 - w1
    idx_ref[...] = jnp.where(ii == 0, i1, jnp.where(ii == 1, i2, 0))
    wt_ref[...] = jnp.where(ii == 0, w1, jnp.where(ii == 1, w2, 0.0))
    ii128 = lax.broadcasted_iota(jnp.int32, (_TOK_BLK, 128), 1)
    ind = ((ii128 == i1) | (ii128 == i2)).astype(jnp.float32)
    gsel = (lax.broadcasted_iota(jnp.int32, (_GPB, _TOK_BLK), 1) // TPW
            == lax.broadcasted_iota(jnp.int32, (_GPB, _TOK_BLK), 0)
            ).astype(jnp.float32)
    cnt = lax.dot_general(gsel, ind, (((1,), (0,)), ((), ())),
                          preferred_element_type=jnp.float32)
    cnt_ref[0] = cnt.astype(jnp.int32)


def _router(x, Wr, br2):
    return pl.pallas_call(
        _router_body,
        grid=(S // _TOK_BLK,),
        in_specs=[
            pl.BlockSpec((_TOK_BLK, EM), lambda i: (i, 0)),
            pl.BlockSpec((NE, EM), lambda i: (0, 0)),
            pl.BlockSpec((1, NE), lambda i: (0, 0)),
        ],
        out_specs=[
            pl.BlockSpec((_TOK_BLK, NE), lambda i: (i, 0)),
            pl.BlockSpec((_TOK_BLK, NE), lambda i: (i, 0)),
            pl.BlockSpec((1, _GPB, 128), lambda i: (i, 0, 0)),
        ],
        out_shape=[
            jax.ShapeDtypeStruct((S, NE), jnp.int32),
            jax.ShapeDtypeStruct((S, NE), jnp.float32),
            jax.ShapeDtypeStruct((S // _TOK_BLK, _GPB, 128), jnp.int32),
        ],
    )(x, Wr, br2)


@functools.cache
def _sc_mesh():
    return plsc.VectorSubcoreMesh(core_axis_name="c", subcore_axis_name="s")


_LANE = lambda: lax.iota(jnp.int32, 16)


def _vsum16(v):
    lane = _LANE()
    for sh in (8, 4, 2, 1):
        v = v + v[lane ^ sh]
    return v


def _cumsum16(v):
    lane = _LANE()
    for sh in (1, 2, 4, 8):
        v = v + jnp.where(lane >= sh, v[jnp.maximum(lane - sh, 0)], 0)
    return v


def _dispatch_body(fe0_hbm, fe1_hbm, cnt_hbm, x_hbm, xs_hbm, pos0_hbm,
                   pos1_hbm, te_hbm, ids_v, cnt_v, xv, idx0_v, idx1_v,
                   te_v, sem):
    c = lax.axis_index("c")
    s = lax.axis_index("s")
    wid = s * 2 + c
    lane = _LANE()
    zero16 = jnp.zeros((16,), jnp.int32)

    pltpu.sync_copy(fe0_hbm.at[pl.ds(wid * TPW, TPW)], ids_v.at[pl.ds(0, TPW)])
    pltpu.sync_copy(fe1_hbm.at[pl.ds(wid * TPW, TPW)], ids_v.at[pl.ds(TPW, TPW)])
    pltpu.sync_copy(cnt_hbm, cnt_v)

    def acc_rows(lo, hi, init):
        def body(i, acc):
            return acc + cnt_v[pl.ds(i * 16, 16)]
        return lax.fori_loop(lo, hi, body, init, unroll=False)

    before = acc_rows(0, wid, zero16)
    total = acc_rows(wid, NW, before)

    aligned = (total + (T - 1)) & jnp.int32(-T)
    incl = _cumsum16(aligned)
    ebase = incl - aligned
    mybase = ebase + before

    running = mybase
    mpw = TPW // 16
    for j in range(2 * mpw):
        idsv = ids_v[pl.ds(j * 16, 16)]
        posv = zero16
        for e in range(NE):
            m = idsv == e
            inc = _cumsum16(jnp.where(m, 1, 0))
            base_e = _vsum16(jnp.where(lane == e, running, 0))
            posv = jnp.where(m, base_e + inc - 1, posv)
            cnt_e = inc[jnp.full((16,), 15, jnp.int32)]
            running = running + jnp.where(lane == e, cnt_e, 0)
        tgt = idx0_v if j < mpw else idx1_v
        tgt[pl.ds((j % mpw) * 16, 16)] = posv

    pltpu.sync_copy(idx0_v, pos0_hbm.at[pl.ds(wid * TPW, TPW)])
    pltpu.sync_copy(idx1_v, pos1_hbm.at[pl.ds(wid * TPW, TPW)])

    pltpu.sync_copy(x_hbm.at[pl.ds(wid * TPW, TPW)], xv)
    pltpu.async_copy(xv, xs_hbm.at[idx0_v], sem).wait()
    pltpu.async_copy(xv, xs_hbm.at[idx1_v], sem).wait()

    @pl.when(wid == 0)
    def _():
        tsv = lax.shift_right_logical(ebase, _TSHIFT)
        ntv = lax.shift_right_logical(aligned, _TSHIFT)
        pf = jnp.where(lane < NE, tsv, ntv[jnp.maximum(lane - NE, 0)])
        te_v[...] = pf
        pltpu.sync_copy(te_v, te_hbm)


@functools.cache
def _dispatch_kernel():
    return pl.kernel(
        _dispatch_body,
        mesh=_sc_mesh(),
        out_type=(
            jax.ShapeDtypeStruct((NTT + XSPAD, EM), jnp.float32),
            jax.ShapeDtypeStruct((S,), jnp.int32),
            jax.ShapeDtypeStruct((S,), jnp.int32),
            jax.ShapeDtypeStruct((16,), jnp.int32),
        ),
        scratch_types=[
            pltpu.VMEM((2 * TPW,), jnp.int32),
            pltpu.VMEM((NW * 16,), jnp.int32),
            pltpu.VMEM((TPW, EM), jnp.float32),
            pltpu.VMEM((TPW,), jnp.int32),
            pltpu.VMEM((TPW,), jnp.int32),
            pltpu.VMEM((16,), jnp.int32),
            pltpu.SemaphoreType.DMA,
        ],
    )


def _dispatch(fe0, fe1, cnt_flat, x):
    return _dispatch_kernel()(fe0, fe1, cnt_flat, x)


HK = HID // 2
NH = HID // HK
MAXT = S // T


def _mm_body(pf_ref, xs_hbm, w1_ref, b1_ref, w2_ref, b2_ref, ys_hbm,
             acc_ref, xbig, ybuf, insem, outsem):
    e = pl.program_id(0)
    k = pl.program_id(1)
    ts = pf_ref[e]
    ntl = pf_ref[NE + e]

    def in_chunk_copy(c):
        return pltpu.make_async_copy(
            xs_hbm.at[pl.ds((ts + c * CH_T) * T, CH_T * T), :],
            xbig.at[pl.ds(c * CH_T * T, CH_T * T), :], insem)

    def out_copy(j, slot):
        return pltpu.make_async_copy(
            ybuf.at[slot], ys_hbm.at[pl.ds((ts + j) * T, 2 * T), :], outsem)

    npair = (ntl + 1) // 2

    @pl.when(k == 0)
    def _():
        nch = (ntl + CH_T - 1) // CH_T

        def fire(c, carry):
            in_chunk_copy(c).start()
            return carry

        lax.fori_loop(0, nch, fire, 0, unroll=False)

    def body(jp, carry):
        j = 2 * jp

        @pl.when((k == 0) & (lax.rem(jp, 2) == 0))
        def _():
            in_chunk_copy(jp // 2).wait()

        xs = xbig[pl.ds(j * T, 2 * T), :]
        h = lax.dot_general(xs, w1_ref[0, 0], (((1,), (1,)), ((), ())),
                            preferred_element_type=jnp.float32)
        h = jnp.maximum(h + b1_ref[0, 0, 0], 0.0)
        y = lax.dot_general(h, w2_ref[0], (((1,), (1,)), ((), ())),
                            preferred_element_type=jnp.float32)

        @pl.when(k == 0)
        def _():
            acc_ref[pl.ds(j * T, 2 * T), :] = y

        @pl.when(k == NH - 1)
        def _():
            oslot = lax.rem(jp, 2)

            @pl.when(jp >= 2)
            def _():
                out_copy(0, oslot).wait()

            ybuf[oslot] = acc_ref[pl.ds(j * T, 2 * T), :] + y + b2_ref[0, 0]
            out_copy(j, oslot).start()

        return carry

    lax.fori_loop(0, npair, body, 0, unroll=False)

    @pl.when(k == NH - 1)
    def _():
        @pl.when(npair >= 1)
        def _():
            out_copy(0, 0).wait()

        @pl.when(npair >= 2)
        def _():
            out_copy(0, 1).wait()


def _mm(pf, xs, W1, b1, W2, b2):
    grid_spec = pltpu.PrefetchScalarGridSpec(
        num_scalar_prefetch=1,
        grid=(NE, NH),
        in_specs=[
            pl.BlockSpec(memory_space=pl.ANY),
            pl.BlockSpec((1, 1, HK, EM), lambda e, k, pf: (e, k, 0, 0)),
            pl.BlockSpec((1, 1, 1, HK), lambda e, k, pf: (e, k, 0, 0)),
            pl.BlockSpec((1, EM, HK), lambda e, k, pf: (e, 0, k)),
            pl.BlockSpec((1, 1, EM), lambda e, k, pf: (e, 0, 0)),
        ],
        out_specs=pl.BlockSpec(memory_space=pl.ANY),
        scratch_shapes=[
            pltpu.VMEM((MAXT * T, EM), jnp.float32),
            pltpu.VMEM((MAXT * T, EM), jnp.float32),
            pltpu.VMEM((2, 2 * T, EM), jnp.float32),
            pltpu.SemaphoreType.DMA,
            pltpu.SemaphoreType.DMA,
        ],
    )
    return pl.pallas_call(
        _mm_body,
        grid_spec=grid_spec,
        out_shape=jax.ShapeDtypeStruct((NTT + T, EM), jnp.float32),
        compiler_params=pltpu.CompilerParams(
            dimension_semantics=("arbitrary", "arbitrary"),
        ),
    )(pf, xs, W1.reshape(NE, NH, HK, EM), b1.reshape(NE, NH, 1, HK),
      W2, b2.reshape(NE, 1, EM))


def _combine_body(ys_hbm, pos0_hbm, pos1_hbm, w0_hbm, w1_hbm, out_hbm,
                  av, bv, i0v, i1v, w0v, w1v, sem):
    c = lax.axis_index("c")
    s = lax.axis_index("s")
    wid = s * 2 + c

    nch = TPW // CHT

    pltpu.sync_copy(pos0_hbm.at[pl.ds(wid * TPW, TPW)], i0v)
    pltpu.sync_copy(pos1_hbm.at[pl.ds(wid * TPW, TPW)], i1v)
    pltpu.sync_copy(w0_hbm.at[pl.ds(wid * TPW, TPW)], w0v)
    pltpu.sync_copy(w1_hbm.at[pl.ds(wid * TPW, TPW)], w1v)

    def fire(ch):
        slot = ch % 2
        pltpu.async_copy(ys_hbm.at[i0v.at[pl.ds(ch * CHT, CHT)]],
                         av.at[slot], sem)
        pltpu.async_copy(ys_hbm.at[i1v.at[pl.ds(ch * CHT, CHT)]],
                         bv.at[slot], sem)

    def drain(ch):
        slot = ch % 2
        pltpu.make_async_copy(ys_hbm.at[i0v.at[pl.ds(ch * CHT, CHT)]],
                              av.at[slot], sem).wait()
        pltpu.make_async_copy(ys_hbm.at[i1v.at[pl.ds(ch * CHT, CHT)]],
                              bv.at[slot], sem).wait()

    fire(0)
    if nch > 1:
        fire(1)
    for ch in range(nch):
        slot = ch % 2
        drain(ch)
        w0c = w0v[pl.ds(ch * CHT, 16)]
        w1c = w1v[pl.ds(ch * CHT, 16)]
        for l in range(CHT):
            wa = w0c[l]
            wb = w1c[l]

            def l_body(k, __, l=l, wa=wa, wb=wb, slot=slot):
                arow = av[slot, l, pl.ds(k * 16, 16)]
                brow = bv[slot, l, pl.ds(k * 16, 16)]
                av[slot, l, pl.ds(k * 16, 16)] = arow * wa + brow * wb
                return 0

            lax.fori_loop(0, EM // 16, l_body, 0, unroll=8)

        pltpu.sync_copy(av.at[slot],
                        out_hbm.at[pl.ds(wid * TPW + ch * CHT, CHT)])
        if ch + 2 < nch:
            fire(ch + 2)


@functools.cache
def _combine_kernel():
    return pl.kernel(
        _combine_body,
        mesh=_sc_mesh(),
        out_type=jax.ShapeDtypeStruct((S, EM), jnp.float32),
        scratch_types=[
            pltpu.VMEM((2, CHT, EM), jnp.float32),
            pltpu.VMEM((2, CHT, EM), jnp.float32),
            pltpu.VMEM((TPW,), jnp.int32),
            pltpu.VMEM((TPW,), jnp.int32),
            pltpu.VMEM((TPW,), jnp.float32),
            pltpu.VMEM((TPW,), jnp.float32),
            pltpu.SemaphoreType.DMA,
        ],
    )


def _combine(ys, pos0, pos1, w0, w1v):
    return _combine_kernel()(ys, pos0, pos1, w0, w1v)


def kernel(inputs, Wr, br, W1, b1, W2, b2):
    x = inputs.reshape(S, EM)
    idx, wt, cnt = _router(x, Wr, br.reshape(1, NE))
    fe0 = idx[:, 0]
    fe1 = idx[:, 1]
    w0 = wt[:, 0]
    w1v = wt[:, 1]
    cnt_flat = cnt.reshape(NW, 128)[:, :16].reshape(NW * 16)
    xs, pos0, pos1, pf = _dispatch(fe0, fe1, cnt_flat, x)
    ys = _mm(pf, xs, W1, b1, W2, b2)
    out = _combine(ys, pos0, pos1, w0, w1v)
    return out.reshape(1, S, EM)


# speedup vs baseline: 1.0457x; 1.0457x over previous
import functools

import jax
import jax.numpy as jnp
from jax import lax
from jax.experimental import pallas as pl
from jax.experimental.pallas import tpu as pltpu
from jax.experimental.pallas import tpu_sc as plsc

S = 2048
EM = 1024
NE = 8
HID = 4096
T = 256
_TSHIFT = 8
CH_T = 4
XSPAD = (CH_T - 1) * T
NTT = S * 2 + NE * T
NT = NTT // T
NT_PAD = 48
NW = 32
TPW = S // NW
CHT = 16


_TOK_BLK = 256


_GPB = _TOK_BLK // TPW


def _router_body(x_ref, wr_ref, br_ref, idx_ref, wt_ref, cnt_ref):
    x = x_ref[...]
    wr = wr_ref[...]
    scores = lax.dot_general(x, wr, (((1,), (1,)), ((), ())),
                             preferred_element_type=jnp.float32)
    scores = scores + br_ref[...]
    ii = lax.broadcasted_iota(jnp.int32, (_TOK_BLK, NE), 1)
    m1 = jnp.max(scores, axis=1, keepdims=True)
    i1 = jnp.min(jnp.where(scores == m1, ii, NE), axis=1, keepdims=True)
    masked = jnp.where(ii == i1, -jnp.inf, scores)
    m2 = jnp.max(masked, axis=1, keepdims=True)
    i2 = jnp.min(jnp.where(masked == m2, ii, NE), axis=1, keepdims=True)
    e2 = jnp.exp(m2 - m1)
    w1 = 1.0 / (1.0 + e2)
    w2 = 1.0 - w1
    idx_ref[...] = jnp.where(ii == 0, i1, jnp.where(ii == 1, i2, 0))
    wt_ref[...] = jnp.where(ii == 0, w1, jnp.where(ii == 1, w2, 0.0))
    ii128 = lax.broadcasted_iota(jnp.int32, (_TOK_BLK, 128), 1)
    ind = ((ii128 == i1) | (ii128 == i2)).astype(jnp.float32)
    gsel = (lax.broadcasted_iota(jnp.int32, (_GPB, _TOK_BLK), 1) // TPW
            == lax.broadcasted_iota(jnp.int32, (_GPB, _TOK_BLK), 0)
            ).astype(jnp.float32)
    cnt = lax.dot_general(gsel, ind, (((1,), (0,)), ((), ())),
                          preferred_element_type=jnp.float32)
    cnt_ref[0] = cnt.astype(jnp.int32)


def _router(x, Wr, br2):
    return pl.pallas_call(
        _router_body,
        grid=(S // _TOK_BLK,),
        in_specs=[
            pl.BlockSpec((_TOK_BLK, EM), lambda i: (i, 0)),
            pl.BlockSpec((NE, EM), lambda i: (0, 0)),
            pl.BlockSpec((1, NE), lambda i: (0, 0)),
        ],
        out_specs=[
            pl.BlockSpec((_TOK_BLK, NE), lambda i: (i, 0)),
            pl.BlockSpec((_TOK_BLK, NE), lambda i: (i, 0)),
            pl.BlockSpec((1, _GPB, 128), lambda i: (i, 0, 0)),
        ],
        out_shape=[
            jax.ShapeDtypeStruct((S, NE), jnp.int32),
            jax.ShapeDtypeStruct((S, NE), jnp.float32),
            jax.ShapeDtypeStruct((S // _TOK_BLK, _GPB, 128), jnp.int32),
        ],
    )(x, Wr, br2)


@functools.cache
def _sc_mesh():
    return plsc.VectorSubcoreMesh(core_axis_name="c", subcore_axis_name="s")


_LANE = lambda: lax.iota(jnp.int32, 16)


def _vsum16(v):
    lane = _LANE()
    for sh in (8, 4, 2, 1):
        v = v + v[lane ^ sh]
    return v


def _cumsum16(v):
    lane = _LANE()
    for sh in (1, 2, 4, 8):
        v = v + jnp.where(lane >= sh, v[jnp.maximum(lane - sh, 0)], 0)
    return v


def _dispatch_body(fe0_hbm, fe1_hbm, cnt_hbm, x_hbm, xs_hbm, pos0_hbm,
                   pos1_hbm, te_hbm, ids_v, cnt_v, xv, idx0_v, idx1_v,
                   te_v, sem):
    c = lax.axis_index("c")
    s = lax.axis_index("s")
    wid = s * 2 + c
    lane = _LANE()
    zero16 = jnp.zeros((16,), jnp.int32)

    pltpu.sync_copy(fe0_hbm.at[pl.ds(wid * TPW, TPW)], ids_v.at[pl.ds(0, TPW)])
    pltpu.sync_copy(fe1_hbm.at[pl.ds(wid * TPW, TPW)], ids_v.at[pl.ds(TPW, TPW)])
    pltpu.sync_copy(cnt_hbm, cnt_v)

    def acc_rows(lo, hi, init):
        def body(i, acc):
            return acc + cnt_v[pl.ds(i * 16, 16)]
        return lax.fori_loop(lo, hi, body, init, unroll=False)

    before = acc_rows(0, wid, zero16)
    total = acc_rows(wid, NW, before)

    aligned = (total + (T - 1)) & jnp.int32(-T)
    incl = _cumsum16(aligned)
    ebase = incl - aligned
    mybase = ebase + before

    running = mybase
    mpw = TPW // 16
    for j in range(2 * mpw):
        idsv = ids_v[pl.ds(j * 16, 16)]
        posv = zero16
        for e in range(NE):
            m = idsv == e
            inc = _cumsum16(jnp.where(m, 1, 0))
            base_e = _vsum16(jnp.where(lane == e, running, 0))
            posv = jnp.where(m, base_e + inc - 1, posv)
            cnt_e = inc[jnp.full((16,), 15, jnp.int32)]
            running = running + jnp.where(lane == e, cnt_e, 0)
        tgt = idx0_v if j < mpw else idx1_v
        tgt[pl.ds((j % mpw) * 16, 16)] = posv

    pltpu.sync_copy(idx0_v, pos0_hbm.at[pl.ds(wid * TPW, TPW)])
    pltpu.sync_copy(idx1_v, pos1_hbm.at[pl.ds(wid * TPW, TPW)])

    pltpu.sync_copy(x_hbm.at[pl.ds(wid * TPW, TPW)], xv)
    pltpu.async_copy(xv, xs_hbm.at[idx0_v], sem).wait()
    pltpu.async_copy(xv, xs_hbm.at[idx1_v], sem).wait()

    @pl.when(wid == 0)
    def _():
        tsv = lax.shift_right_logical(ebase, _TSHIFT)
        ntv = lax.shift_right_logical(aligned, _TSHIFT)
        pf = jnp.where(lane < NE, tsv, ntv[jnp.maximum(lane - NE, 0)])
        te_v[...] = pf
        pltpu.sync_copy(te_v, te_hbm)


@functools.cache
def _dispatch_kernel():
    return pl.kernel(
        _dispatch_body,
        mesh=_sc_mesh(),
        out_type=(
            jax.ShapeDtypeStruct((NTT + XSPAD, EM), jnp.float32),
            jax.ShapeDtypeStruct((S,), jnp.int32),
            jax.ShapeDtypeStruct((S,), jnp.int32),
            jax.ShapeDtypeStruct((16,), jnp.int32),
        ),
        scratch_types=[
            pltpu.VMEM((2 * TPW,), jnp.int32),
            pltpu.VMEM((NW * 16,), jnp.int32),
            pltpu.VMEM((TPW, EM), jnp.float32),
            pltpu.VMEM((TPW,), jnp.int32),
            pltpu.VMEM((TPW,), jnp.int32),
            pltpu.VMEM((16,), jnp.int32),
            pltpu.SemaphoreType.DMA,
        ],
    )


def _dispatch(fe0, fe1, cnt_flat, x):
    return _dispatch_kernel()(fe0, fe1, cnt_flat, x)


HK = HID // 2
NH = HID // HK
MAXT = S // T


def _mm_body(pf_ref, xs_hbm, w1_ref, b1_ref, w2_ref, b2_ref, ys_hbm,
             acc_ref, xbig, ybuf, insem, outsem):
    e = pl.program_id(0)
    k = pl.program_id(1)
    ts = pf_ref[e]
    ntl = pf_ref[NE + e]

    def in_chunk_copy(c):
        return pltpu.make_async_copy(
            xs_hbm.at[pl.ds((ts + c * CH_T) * T, CH_T * T), :],
            xbig.at[pl.ds(c * CH_T * T, CH_T * T), :], insem)

    def out_copy(j, slot):
        return pltpu.make_async_copy(
            ybuf.at[slot], ys_hbm.at[pl.ds((ts + j) * T, T), :], outsem)

    @pl.when(k == 0)
    def _():
        nch = (ntl + CH_T - 1) // CH_T

        def fire(c, carry):
            in_chunk_copy(c).start()
            return carry

        lax.fori_loop(0, nch, fire, 0, unroll=False)

    def body(j, carry):
        @pl.when((k == 0) & (lax.rem(j, CH_T) == 0))
        def _():
            in_chunk_copy(j // CH_T).wait()

        xs = xbig[pl.ds(j * T, T), :]
        h = lax.dot_general(xs, w1_ref[0, 0], (((1,), (1,)), ((), ())),
                            preferred_element_type=jnp.float32)
        h = jnp.maximum(h + b1_ref[0, 0, 0], 0.0)
        y = lax.dot_general(h, w2_ref[0], (((1,), (1,)), ((), ())),
                            preferred_element_type=jnp.float32)

        @pl.when(k == 0)
        def _():
            acc_ref[pl.ds(j * T, T), :] = y

        @pl.when(k == NH - 1)
        def _():
            oslot = lax.rem(j, 2)

            @pl.when(j >= 2)
            def _():
                out_copy(0, oslot).wait()

            ybuf[oslot] = acc_ref[pl.ds(j * T, T), :] + y + b2_ref[0, 0]
            out_copy(j, oslot).start()

        return carry

    lax.fori_loop(0, ntl, body, 0, unroll=False)

    @pl.when(k == NH - 1)
    def _():
        @pl.when(ntl >= 1)
        def _():
            out_copy(0, 0).wait()

        @pl.when(ntl >= 2)
        def _():
            out_copy(0, 1).wait()


def _mm(pf, xs, W1, b1, W2, b2):
    grid_spec = pltpu.PrefetchScalarGridSpec(
        num_scalar_prefetch=1,
        grid=(NE, NH),
        in_specs=[
            pl.BlockSpec(memory_space=pl.ANY),
            pl.BlockSpec((1, 1, HK, EM), lambda e, k, pf: (e, k, 0, 0)),
            pl.BlockSpec((1, 1, 1, HK), lambda e, k, pf: (e, k, 0, 0)),
            pl.BlockSpec((1, EM, HK), lambda e, k, pf: (e, 0, k)),
            pl.BlockSpec((1, 1, EM), lambda e, k, pf: (e, 0, 0)),
        ],
        out_specs=pl.BlockSpec(memory_space=pl.ANY),
        scratch_shapes=[
            pltpu.VMEM((MAXT * T, EM), jnp.float32),
            pltpu.VMEM((MAXT * T, EM), jnp.float32),
            pltpu.VMEM((2, T, EM), jnp.float32),
            pltpu.SemaphoreType.DMA,
            pltpu.SemaphoreType.DMA,
        ],
    )
    return pl.pallas_call(
        _mm_body,
        grid_spec=grid_spec,
        out_shape=jax.ShapeDtypeStruct((NTT + T, EM), jnp.float32),
        compiler_params=pltpu.CompilerParams(
            dimension_semantics=("arbitrary", "arbitrary"),
        ),
    )(pf, xs, W1.reshape(NE, NH, HK, EM), b1.reshape(NE, NH, 1, HK),
      W2, b2.reshape(NE, 1, EM))


def _combine_body(ys_hbm, pos0_hbm, pos1_hbm, w0_hbm, w1_hbm, out_hbm,
                  av, bv, i0v, i1v, w0v, w1v, sem):
    c = lax.axis_index("c")
    s = lax.axis_index("s")
    wid = s * 2 + c

    nch = TPW // CHT

    pltpu.sync_copy(pos0_hbm.at[pl.ds(wid * TPW, TPW)], i0v)
    pltpu.sync_copy(pos1_hbm.at[pl.ds(wid * TPW, TPW)], i1v)
    pltpu.sync_copy(w0_hbm.at[pl.ds(wid * TPW, TPW)], w0v)
    pltpu.sync_copy(w1_hbm.at[pl.ds(wid * TPW, TPW)], w1v)

    def fire(ch):
        slot = ch % 2
        pltpu.async_copy(ys_hbm.at[i0v.at[pl.ds(ch * CHT, CHT)]],
                         av.at[slot], sem)
        pltpu.async_copy(ys_hbm.at[i1v.at[pl.ds(ch * CHT, CHT)]],
                         bv.at[slot], sem)

    def drain(ch):
        slot = ch % 2
        pltpu.make_async_copy(ys_hbm.at[i0v.at[pl.ds(ch * CHT, CHT)]],
                              av.at[slot], sem).wait()
        pltpu.make_async_copy(ys_hbm.at[i1v.at[pl.ds(ch * CHT, CHT)]],
                              bv.at[slot], sem).wait()

    fire(0)
    if nch > 1:
        fire(1)
    for ch in range(nch):
        slot = ch % 2
        drain(ch)
        w0c = w0v[pl.ds(ch * CHT, 16)]
        w1c = w1v[pl.ds(ch * CHT, 16)]
        for l in range(CHT):
            wa = w0c[l]
            wb = w1c[l]

            def l_body(k, __, l=l, wa=wa, wb=wb, slot=slot):
                arow = av[slot, l, pl.ds(k * 16, 16)]
                brow = bv[slot, l, pl.ds(k * 16, 16)]
                av[slot, l, pl.ds(k * 16, 16)] = arow * wa + brow * wb
                return 0

            lax.fori_loop(0, EM // 16, l_body, 0, unroll=8)

        pltpu.sync_copy(av.at[slot],
                        out_hbm.at[pl.ds(wid * TPW + ch * CHT, CHT)])
        if ch + 2 < nch:
            fire(ch + 2)


@functools.cache
def _combine_kernel():
    return pl.kernel(
        _combine_body,
        mesh=_sc_mesh(),
        out_type=jax.ShapeDtypeStruct((S, EM), jnp.float32),
        scratch_types=[
            pltpu.VMEM((2, CHT, EM), jnp.float32),
            pltpu.VMEM((2, CHT, EM), jnp.float32),
            pltpu.VMEM((TPW,), jnp.int32),
            pltpu.VMEM((TPW,), jnp.int32),
            pltpu.VMEM((TPW,), jnp.float32),
            pltpu.VMEM((TPW,), jnp.float32),
            pltpu.SemaphoreType.DMA,
        ],
    )


def _combine(ys, pos0, pos1, w0, w1v):
    return _combine_kernel()(ys, pos0, pos1, w0, w1v)


def kernel(inputs, Wr, br, W1, b1, W2, b2):
    x = inputs.reshape(S, EM)
    idx, wt, cnt = _router(x, Wr, br.reshape(1, NE))
    fe0 = idx[:, 0]
    fe1 = idx[:, 1]
    w0 = wt[:, 0]
    w1v = wt[:, 1]
    cnt_flat = cnt.reshape(NW, 128)[:, :16].reshape(NW * 16)
    xs, pos0, pos1, pf = _dispatch(fe0, fe1, cnt_flat, x)
    ys = _mm(pf, xs, W1, b1, W2, b2)
    out = _combine(ys, pos0, pos1, w0, w1v)
    return out.reshape(1, S, EM)


# speedup vs baseline: 1.0592x; 1.0129x over previous
import functools

import jax
import jax.numpy as jnp
from jax import lax
from jax.experimental import pallas as pl
from jax.experimental.pallas import tpu as pltpu
from jax.experimental.pallas import tpu_sc as plsc

S = 2048
EM = 1024
NE = 8
HID = 4096
T = 256
_TSHIFT = 8
CH_T = 4
XSPAD = (CH_T - 1) * T
NTT = S * 2 + NE * T
NT = NTT // T
NT_PAD = 48
NW = 32
TPW = S // NW
CHT = 16


_TOK_BLK = 512


_GPB = _TOK_BLK // TPW


def _router_body(x_ref, wr_ref, br_ref, idx_ref, wt_ref, cnt_ref):
    x = x_ref[...]
    wr = wr_ref[...]
    scores = lax.dot_general(x, wr, (((1,), (1,)), ((), ())),
                             preferred_element_type=jnp.float32)
    scores = scores + br_ref[...]
    ii = lax.broadcasted_iota(jnp.int32, (_TOK_BLK, NE), 1)
    m1 = jnp.max(scores, axis=1, keepdims=True)
    i1 = jnp.min(jnp.where(scores == m1, ii, NE), axis=1, keepdims=True)
    masked = jnp.where(ii == i1, -jnp.inf, scores)
    m2 = jnp.max(masked, axis=1, keepdims=True)
    i2 = jnp.min(jnp.where(masked == m2, ii, NE), axis=1, keepdims=True)
    e2 = jnp.exp(m2 - m1)
    w1 = 1.0 / (1.0 + e2)
    w2 = 1.0 - w1
    idx_ref[...] = jnp.where(ii == 0, i1, jnp.where(ii == 1, i2, 0))
    wt_ref[...] = jnp.where(ii == 0, w1, jnp.where(ii == 1, w2, 0.0))
    ii128 = lax.broadcasted_iota(jnp.int32, (_TOK_BLK, 128), 1)
    ind = ((ii128 == i1) | (ii128 == i2)).astype(jnp.float32)
    gsel = (lax.broadcasted_iota(jnp.int32, (_GPB, _TOK_BLK), 1) // TPW
            == lax.broadcasted_iota(jnp.int32, (_GPB, _TOK_BLK), 0)
            ).astype(jnp.float32)
    cnt = lax.dot_general(gsel, ind, (((1,), (0,)), ((), ())),
                          preferred_element_type=jnp.float32)
    cnt_ref[0] = cnt.astype(jnp.int32)


def _router(x, Wr, br2):
    return pl.pallas_call(
        _router_body,
        grid=(S // _TOK_BLK,),
        in_specs=[
            pl.BlockSpec((_TOK_BLK, EM), lambda i: (i, 0)),
            pl.BlockSpec((NE, EM), lambda i: (0, 0)),
            pl.BlockSpec((1, NE), lambda i: (0, 0)),
        ],
        out_specs=[
            pl.BlockSpec((_TOK_BLK, NE), lambda i: (i, 0)),
            pl.BlockSpec((_TOK_BLK, NE), lambda i: (i, 0)),
            pl.BlockSpec((1, _GPB, 128), lambda i: (i, 0, 0)),
        ],
        out_shape=[
            jax.ShapeDtypeStruct((S, NE), jnp.int32),
            jax.ShapeDtypeStruct((S, NE), jnp.float32),
            jax.ShapeDtypeStruct((S // _TOK_BLK, _GPB, 128), jnp.int32),
        ],
    )(x, Wr, br2)


@functools.cache
def _sc_mesh():
    return plsc.VectorSubcoreMesh(core_axis_name="c", subcore_axis_name="s")


_LANE = lambda: lax.iota(jnp.int32, 16)


def _vsum16(v):
    lane = _LANE()
    for sh in (8, 4, 2, 1):
        v = v + v[lane ^ sh]
    return v


def _cumsum16(v):
    lane = _LANE()
    for sh in (1, 2, 4, 8):
        v = v + jnp.where(lane >= sh, v[jnp.maximum(lane - sh, 0)], 0)
    return v


def _dispatch_body(fe0_hbm, fe1_hbm, cnt_hbm, x_hbm, xs_hbm, pos0_hbm,
                   pos1_hbm, te_hbm, ids_v, cnt_v, xv, idx0_v, idx1_v,
                   te_v, sem):
    c = lax.axis_index("c")
    s = lax.axis_index("s")
    wid = s * 2 + c
    lane = _LANE()
    zero16 = jnp.zeros((16,), jnp.int32)

    pltpu.sync_copy(fe0_hbm.at[pl.ds(wid * TPW, TPW)], ids_v.at[pl.ds(0, TPW)])
    pltpu.sync_copy(fe1_hbm.at[pl.ds(wid * TPW, TPW)], ids_v.at[pl.ds(TPW, TPW)])
    pltpu.sync_copy(cnt_hbm, cnt_v)

    def acc_rows(lo, hi, init):
        def body(i, acc):
            return acc + cnt_v[pl.ds(i * 16, 16)]
        return lax.fori_loop(lo, hi, body, init, unroll=False)

    before = acc_rows(0, wid, zero16)
    total = acc_rows(wid, NW, before)

    aligned = (total + (T - 1)) & jnp.int32(-T)
    incl = _cumsum16(aligned)
    ebase = incl - aligned
    mybase = ebase + before

    running = mybase
    mpw = TPW // 16
    for j in range(2 * mpw):
        idsv = ids_v[pl.ds(j * 16, 16)]
        posv = zero16
        for e in range(NE):
            m = idsv == e
            inc = _cumsum16(jnp.where(m, 1, 0))
            base_e = _vsum16(jnp.where(lane == e, running, 0))
            posv = jnp.where(m, base_e + inc - 1, posv)
            cnt_e = inc[jnp.full((16,), 15, jnp.int32)]
            running = running + jnp.where(lane == e, cnt_e, 0)
        tgt = idx0_v if j < mpw else idx1_v
        tgt[pl.ds((j % mpw) * 16, 16)] = posv

    pltpu.sync_copy(idx0_v, pos0_hbm.at[pl.ds(wid * TPW, TPW)])
    pltpu.sync_copy(idx1_v, pos1_hbm.at[pl.ds(wid * TPW, TPW)])

    pltpu.sync_copy(x_hbm.at[pl.ds(wid * TPW, TPW)], xv)
    pltpu.async_copy(xv, xs_hbm.at[idx0_v], sem).wait()
    pltpu.async_copy(xv, xs_hbm.at[idx1_v], sem).wait()

    @pl.when(wid == 0)
    def _():
        tsv = lax.shift_right_logical(ebase, _TSHIFT)
        ntv = lax.shift_right_logical(aligned, _TSHIFT)
        pf = jnp.where(lane < NE, tsv, ntv[jnp.maximum(lane - NE, 0)])
        te_v[...] = pf
        pltpu.sync_copy(te_v, te_hbm)


@functools.cache
def _dispatch_kernel():
    return pl.kernel(
        _dispatch_body,
        mesh=_sc_mesh(),
        out_type=(
            jax.ShapeDtypeStruct((NTT + XSPAD, EM), jnp.float32),
            jax.ShapeDtypeStruct((S,), jnp.int32),
            jax.ShapeDtypeStruct((S,), jnp.int32),
            jax.ShapeDtypeStruct((16,), jnp.int32),
        ),
        scratch_types=[
            pltpu.VMEM((2 * TPW,), jnp.int32),
            pltpu.VMEM((NW * 16,), jnp.int32),
            pltpu.VMEM((TPW, EM), jnp.float32),
            pltpu.VMEM((TPW,), jnp.int32),
            pltpu.VMEM((TPW,), jnp.int32),
            pltpu.VMEM((16,), jnp.int32),
            pltpu.SemaphoreType.DMA,
        ],
    )


def _dispatch(fe0, fe1, cnt_flat, x):
    return _dispatch_kernel()(fe0, fe1, cnt_flat, x)


HK = HID // 2
NH = HID // HK
MAXT = S // T


def _mm_body(pf_ref, xs_hbm, w1_ref, b1_ref, w2_ref, b2_ref, ys_hbm,
             acc_ref, xbig, ybuf, insem, outsem):
    e = pl.program_id(0)
    k = pl.program_id(1)
    ts = pf_ref[e]
    ntl = pf_ref[NE + e]

    def in_chunk_copy(c):
        return pltpu.make_async_copy(
            xs_hbm.at[pl.ds((ts + c * CH_T) * T, CH_T * T), :],
            xbig.at[pl.ds(c * CH_T * T, CH_T * T), :], insem)

    def out_copy(j, slot):
        return pltpu.make_async_copy(
            ybuf.at[slot], ys_hbm.at[pl.ds((ts + j) * T, T), :], outsem)

    @pl.when(k == 0)
    def _():
        nch = (ntl + CH_T - 1) // CH_T

        def fire(c, carry):
            in_chunk_copy(c).start()
            return carry

        lax.fori_loop(0, nch, fire, 0, unroll=False)

    def body(j, carry):
        @pl.when((k == 0) & (lax.rem(j, CH_T) == 0))
        def _():
            in_chunk_copy(j // CH_T).wait()

        xs = xbig[pl.ds(j * T, T), :]
        h = lax.dot_general(xs, w1_ref[0, 0], (((1,), (1,)), ((), ())),
                            preferred_element_type=jnp.float32)
        h = jnp.maximum(h + b1_ref[0, 0, 0], 0.0)
        y = lax.dot_general(h, w2_ref[0], (((1,), (1,)), ((), ())),
                            preferred_element_type=jnp.float32)

        @pl.when(k == 0)
        def _():
            acc_ref[pl.ds(j * T, T), :] = y

        @pl.when(k == NH - 1)
        def _():
            oslot = lax.rem(j, 2)

            @pl.when(j >= 2)
            def _():
                out_copy(0, oslot).wait()

            ybuf[oslot] = acc_ref[pl.ds(j * T, T), :] + y + b2_ref[0, 0]
            out_copy(j, oslot).start()

        return carry

    lax.fori_loop(0, ntl, body, 0, unroll=False)

    @pl.when(k == NH - 1)
    def _():
        @pl.when(ntl >= 1)
        def _():
            out_copy(0, 0).wait()

        @pl.when(ntl >= 2)
        def _():
            out_copy(0, 1).wait()


def _mm(pf, xs, W1, b1, W2, b2):
    grid_spec = pltpu.PrefetchScalarGridSpec(
        num_scalar_prefetch=1,
        grid=(NE, NH),
        in_specs=[
            pl.BlockSpec(memory_space=pl.ANY),
            pl.BlockSpec((1, 1, HK, EM), lambda e, k, pf: (e, k, 0, 0)),
            pl.BlockSpec((1, 1, 1, HK), lambda e, k, pf: (e, k, 0, 0)),
            pl.BlockSpec((1, EM, HK), lambda e, k, pf: (e, 0, k)),
            pl.BlockSpec((1, 1, EM), lambda e, k, pf: (e, 0, 0)),
        ],
        out_specs=pl.BlockSpec(memory_space=pl.ANY),
        scratch_shapes=[
            pltpu.VMEM((MAXT * T, EM), jnp.float32),
            pltpu.VMEM((MAXT * T, EM), jnp.float32),
            pltpu.VMEM((2, T, EM), jnp.float32),
            pltpu.SemaphoreType.DMA,
            pltpu.SemaphoreType.DMA,
        ],
    )
    return pl.pallas_call(
        _mm_body,
        grid_spec=grid_spec,
        out_shape=jax.ShapeDtypeStruct((NTT + T, EM), jnp.float32),
        compiler_params=pltpu.CompilerParams(
            dimension_semantics=("arbitrary", "arbitrary"),
        ),
    )(pf, xs, W1.reshape(NE, NH, HK, EM), b1.reshape(NE, NH, 1, HK),
      W2, b2.reshape(NE, 1, EM))


def _combine_body(ys_hbm, pos0_hbm, pos1_hbm, w0_hbm, w1_hbm, out_hbm,
                  av, bv, i0v, i1v, w0v, w1v, sem):
    c = lax.axis_index("c")
    s = lax.axis_index("s")
    wid = s * 2 + c

    nch = TPW // CHT

    pltpu.sync_copy(pos0_hbm.at[pl.ds(wid * TPW, TPW)], i0v)
    pltpu.sync_copy(pos1_hbm.at[pl.ds(wid * TPW, TPW)], i1v)
    pltpu.sync_copy(w0_hbm.at[pl.ds(wid * TPW, TPW)], w0v)
    pltpu.sync_copy(w1_hbm.at[pl.ds(wid * TPW, TPW)], w1v)

    def fire(ch):
        slot = ch % 2
        pltpu.async_copy(ys_hbm.at[i0v.at[pl.ds(ch * CHT, CHT)]],
                         av.at[slot], sem)
        pltpu.async_copy(ys_hbm.at[i1v.at[pl.ds(ch * CHT, CHT)]],
                         bv.at[slot], sem)

    def drain(ch):
        slot = ch % 2
        pltpu.make_async_copy(ys_hbm.at[i0v.at[pl.ds(ch * CHT, CHT)]],
                              av.at[slot], sem).wait()
        pltpu.make_async_copy(ys_hbm.at[i1v.at[pl.ds(ch * CHT, CHT)]],
                              bv.at[slot], sem).wait()

    fire(0)
    if nch > 1:
        fire(1)
    for ch in range(nch):
        slot = ch % 2
        drain(ch)
        w0c = w0v[pl.ds(ch * CHT, 16)]
        w1c = w1v[pl.ds(ch * CHT, 16)]
        for l in range(CHT):
            wa = w0c[l]
            wb = w1c[l]

            def l_body(k, __, l=l, wa=wa, wb=wb, slot=slot):
                arow = av[slot, l, pl.ds(k * 16, 16)]
                brow = bv[slot, l, pl.ds(k * 16, 16)]
                av[slot, l, pl.ds(k * 16, 16)] = arow * wa + brow * wb
                return 0

            lax.fori_loop(0, EM // 16, l_body, 0, unroll=8)

        pltpu.sync_copy(av.at[slot],
                        out_hbm.at[pl.ds(wid * TPW + ch * CHT, CHT)])
        if ch + 2 < nch:
            fire(ch + 2)


@functools.cache
def _combine_kernel():
    return pl.kernel(
        _combine_body,
        mesh=_sc_mesh(),
        out_type=jax.ShapeDtypeStruct((S, EM), jnp.float32),
        scratch_types=[
            pltpu.VMEM((2, CHT, EM), jnp.float32),
            pltpu.VMEM((2, CHT, EM), jnp.float32),
            pltpu.VMEM((TPW,), jnp.int32),
            pltpu.VMEM((TPW,), jnp.int32),
            pltpu.VMEM((TPW,), jnp.float32),
            pltpu.VMEM((TPW,), jnp.float32),
            pltpu.SemaphoreType.DMA,
        ],
    )


def _combine(ys, pos0, pos1, w0, w1v):
    return _combine_kernel()(ys, pos0, pos1, w0, w1v)


def kernel(inputs, Wr, br, W1, b1, W2, b2):
    x = inputs.reshape(S, EM)
    idx, wt, cnt = _router(x, Wr, br.reshape(1, NE))
    fe0 = idx[:, 0]
    fe1 = idx[:, 1]
    w0 = wt[:, 0]
    w1v = wt[:, 1]
    cnt_flat = cnt.reshape(NW, 128)[:, :16].reshape(NW * 16)
    xs, pos0, pos1, pf = _dispatch(fe0, fe1, cnt_flat, x)
    ys = _mm(pf, xs, W1, b1, W2, b2)
    out = _combine(ys, pos0, pos1, w0, w1v)
    return out.reshape(1, S, EM)
